# Initial kernel scaffold; baseline (speedup 1.0000x reference)
import jax
import jax.numpy as jnp
from jax.experimental import pallas as pl


def kernel(x, adj, W1, att_src1, att_dst1, b1, W2, att_src2, att_dst2, b2, fcm_w1, fcm_b1, fcm_w2, fcm_b2, fcl_w1, fcl_b1, fcl_w2, fcl_b2, at_w, at_b, dec_w, dec_b):
    raise NotImplementedError("write your pallas kernel here")


# speedup vs baseline: 2921.1499x; 2921.1499x over previous
import jax
import jax.numpy as jnp
from jax.experimental import pallas as pl

_N = 1024
_IN = 256
_HID = 64
_HEADS = 4
_H4 = _HEADS * _HID
_LAT = 64
_BJ = 256
_BI = 256


def _pre_kernel(x_ref, w_ref, asrc_w_ref, adst_w_ref, h_ref, asrc_ref, adst_ref):
    h = jnp.dot(x_ref[...], w_ref[...], preferred_element_type=jnp.float32)
    h_ref[...] = h
    asrc_ref[...] = jnp.dot(h, asrc_w_ref[...], preferred_element_type=jnp.float32)
    adst_ref[...] = jnp.dot(h, adst_w_ref[...], preferred_element_type=jnp.float32)


def _agg_kernel(adjT_ref, h_ref, asrcT_ref, adst_ref, b_ref, out_ref):
    maskT = adjT_ref[...] != 0.0
    h = h_ref[...]
    cols = []
    for k in range(_HEADS):
        e = adst_ref[:, k : k + 1] + asrcT_ref[k : k + 1, :]
        e = jnp.where(e > 0, e, 0.2 * e)
        e = jnp.where(maskT, e, -jnp.inf)
        m = jnp.max(e, axis=1, keepdims=True)
        m = jnp.where(m == -jnp.inf, 0.0, m)
        p = jnp.exp(e - m)
        s = jnp.sum(p, axis=1, keepdims=True) + 1e-16
        pn = p / s
        cols.append(
            jnp.dot(pn, h[:, _HID * k : _HID * (k + 1)],
                    preferred_element_type=jnp.float32)
        )
    out = jnp.concatenate(cols, axis=1) + b_ref[...]
    out_ref[...] = jnp.maximum(out, 0.0)


def _head_kernel(h_ref, fm1_ref, fmb1_ref, fm2_ref, fmb2_ref,
                 fl1_ref, flb1_ref, fl2_ref, flb2_ref,
                 atw_ref, atb_ref, eps_ref, dwa_ref, dwb_ref,
                 mean_ref, logvar_ref, u_ref, v_ref):
    h = h_ref[...]
    t = jnp.maximum(
        jnp.dot(h, fm1_ref[...], preferred_element_type=jnp.float32)
        + fmb1_ref[...], 0.0)
    mean = jnp.dot(t, fm2_ref[...], preferred_element_type=jnp.float32) + fmb2_ref[...]
    t2 = jnp.maximum(
        jnp.dot(h, fl1_ref[...], preferred_element_type=jnp.float32)
        + flb1_ref[...], 0.0)
    logvar = jnp.dot(t2, fl2_ref[...], preferred_element_type=jnp.float32) + flb2_ref[...]
    std = jnp.exp(0.5 * logvar)
    z = mean + eps_ref[...] * std
    zt = jnp.maximum(
        jnp.dot(z, atw_ref[...], preferred_element_type=jnp.float32)
        + atb_ref[...], 0.0)
    mean_ref[...] = mean
    logvar_ref[...] = logvar
    u_ref[...] = jnp.dot(zt, dwa_ref[...], preferred_element_type=jnp.float32)
    v_ref[...] = jnp.dot(zt, dwb_ref[...], preferred_element_type=jnp.float32)


def _dec_kernel(u_ref, vT_ref, db_ref, out_ref):
    i0 = pl.program_id(0) * _BI
    logits = u_ref[...] + vT_ref[...] + db_ref[0, 0]
    r = jax.nn.sigmoid(logits)
    rows = jax.lax.broadcasted_iota(jnp.int32, (_BI, _N), 0) + i0
    colz = jax.lax.broadcasted_iota(jnp.int32, (_BI, _N), 1)
    out_ref[...] = jnp.where(rows == colz, 0.0, r)


def _blockdiag(att):
    eye = jnp.eye(_HEADS, dtype=att.dtype)
    return (att[:, :, None] * eye[:, None, :]).reshape(_H4, _HEADS)


def _run_pre(x, W, asrc_w, adst_w):
    return pl.pallas_call(
        _pre_kernel,
        out_shape=(
            jax.ShapeDtypeStruct((_N, _H4), jnp.float32),
            jax.ShapeDtypeStruct((_N, _HEADS), jnp.float32),
            jax.ShapeDtypeStruct((_N, _HEADS), jnp.float32),
        ),
    )(x, W, asrc_w, adst_w)


def _run_agg(adjT, h, asrcT, adst, b2d):
    grid = (_N // _BJ,)
    return pl.pallas_call(
        _agg_kernel,
        grid=grid,
        in_specs=[
            pl.BlockSpec((_BJ, _N), lambda j: (j, 0)),
            pl.BlockSpec((_N, _H4), lambda j: (0, 0)),
            pl.BlockSpec((_HEADS, _N), lambda j: (0, 0)),
            pl.BlockSpec((_BJ, _HEADS), lambda j: (j, 0)),
            pl.BlockSpec((1, _H4), lambda j: (0, 0)),
        ],
        out_specs=pl.BlockSpec((_BJ, _H4), lambda j: (j, 0)),
        out_shape=jax.ShapeDtypeStruct((_N, _H4), jnp.float32),
    )(adjT, h, asrcT, adst, b2d)


def kernel(x, adj, W1, att_src1, att_dst1, b1, W2, att_src2, att_dst2, b2,
           fcm_w1, fcm_b1, fcm_w2, fcm_b2, fcl_w1, fcl_b1, fcl_w2, fcl_b2,
           at_w, at_b, dec_w, dec_b):
    adjT = adj.T
    eps = jax.random.normal(jax.random.key(42), (_N, _LAT), dtype=jnp.float32)

    h1, asrc1, adst1 = _run_pre(x, W1, _blockdiag(att_src1), _blockdiag(att_dst1))
    g1 = _run_agg(adjT, h1, asrc1.T, adst1, b1.reshape(1, _H4))

    h2, asrc2, adst2 = _run_pre(g1, W2, _blockdiag(att_src2), _blockdiag(att_dst2))
    g2 = _run_agg(adjT, h2, asrc2.T, adst2, b2.reshape(1, _H4))

    mean, logvar, u, v = pl.pallas_call(
        _head_kernel,
        out_shape=(
            jax.ShapeDtypeStruct((_N, _LAT), jnp.float32),
            jax.ShapeDtypeStruct((_N, _LAT), jnp.float32),
            jax.ShapeDtypeStruct((_N, 1), jnp.float32),
            jax.ShapeDtypeStruct((_N, 1), jnp.float32),
        ),
    )(g2, fcm_w1, fcm_b1.reshape(1, _LAT), fcm_w2, fcm_b2.reshape(1, _LAT),
      fcl_w1, fcl_b1.reshape(1, _LAT), fcl_w2, fcl_b2.reshape(1, _LAT),
      at_w, at_b.reshape(1, _HID), eps,
      dec_w[:_HID].reshape(_HID, 1), dec_w[_HID:].reshape(_HID, 1))

    adj_recon = pl.pallas_call(
        _dec_kernel,
        grid=(_N // _BI,),
        in_specs=[
            pl.BlockSpec((_BI, 1), lambda i: (i, 0)),
            pl.BlockSpec((1, _N), lambda i: (0, 0)),
            pl.BlockSpec((1, 1), lambda i: (0, 0)),
        ],
        out_specs=pl.BlockSpec((_BI, _N), lambda i: (i, 0)),
        out_shape=jax.ShapeDtypeStruct((_N, _N), jnp.float32),
    )(u, v.T, dec_b.reshape(1, 1))

    return adj_recon, mean, logvar


# speedup vs baseline: 4434.6929x; 1.5181x over previous
import jax
import jax.numpy as jnp
from jax.experimental import pallas as pl

_N = 1024
_IN = 256
_HID = 64
_HEADS = 4
_H4 = _HEADS * _HID
_LAT = 64


def _fused_kernel(x_ref, adj_ref,
                  w1_ref, as1_ref, ad1_ref, b1_ref,
                  w2_ref, as2_ref, ad2_ref, b2_ref,
                  fm1_ref, fmb1_ref, fm2_ref, fmb2_ref,
                  fl1_ref, flb1_ref, fl2_ref, flb2_ref,
                  atw_ref, atb_ref, eps_ref, dwa_ref, dwb_ref, db_ref,
                  rec_ref, mean_ref, logvar_ref):
    mask = adj_ref[...] != 0.0
    ones_col = jnp.ones((_N, 1), dtype=jnp.float32)
    neg_inf = jnp.float32(-jnp.inf)

    def gat(inp, w_ref, asw_ref, adw_ref, b_ref):
        h = jnp.dot(inp, w_ref[...], preferred_element_type=jnp.float32)
        asrc = jnp.dot(h, asw_ref[...], preferred_element_type=jnp.float32)
        adstT = jax.lax.dot_general(
            adw_ref[...], h, (((0,), (1,)), ((), ())),
            preferred_element_type=jnp.float32)
        outs = []
        for k in range(_HEADS):
            e = asrc[:, k : k + 1] + adstT[k : k + 1, :]
            e = jnp.where(e > 0, e, 0.2 * e)
            e = jnp.where(mask, e, neg_inf)
            m = jnp.max(e, axis=0, keepdims=True)
            m = jnp.where(m == neg_inf, 0.0, m)
            p = jnp.exp(e - m)
            s = jax.lax.dot_general(p, ones_col, (((0,), (0,)), ((), ())),
                                    preferred_element_type=jnp.float32)
            o = jax.lax.dot_general(
                p, h[:, _HID * k : _HID * (k + 1)], (((0,), (0,)), ((), ())),
                preferred_element_type=jnp.float32)
            outs.append(o * (1.0 / (s + 1e-16)))
        g = jnp.concatenate(outs, axis=1) + b_ref[...]
        return jnp.maximum(g, 0.0)

    g1 = gat(x_ref[...], w1_ref, as1_ref, ad1_ref, b1_ref)
    g2 = gat(g1, w2_ref, as2_ref, ad2_ref, b2_ref)

    t = jnp.maximum(
        jnp.dot(g2, fm1_ref[...], preferred_element_type=jnp.float32)
        + fmb1_ref[...], 0.0)
    mean = jnp.dot(t, fm2_ref[...], preferred_element_type=jnp.float32) + fmb2_ref[...]
    t2 = jnp.maximum(
        jnp.dot(g2, fl1_ref[...], preferred_element_type=jnp.float32)
        + flb1_ref[...], 0.0)
    logvar = jnp.dot(t2, fl2_ref[...], preferred_element_type=jnp.float32) + flb2_ref[...]
    std = jnp.exp(0.5 * logvar)
    z = mean + eps_ref[...] * std
    zt = jnp.maximum(
        jnp.dot(z, atw_ref[...], preferred_element_type=jnp.float32)
        + atb_ref[...], 0.0)
    u = jnp.dot(zt, dwa_ref[...], preferred_element_type=jnp.float32)
    vT = jax.lax.dot_general(
        dwb_ref[...], zt, (((0,), (1,)), ((), ())),
        preferred_element_type=jnp.float32)
    r = jax.nn.sigmoid(u + vT + db_ref[0, 0])
    rows = jax.lax.broadcasted_iota(jnp.int32, (_N, _N), 0)
    colz = jax.lax.broadcasted_iota(jnp.int32, (_N, _N), 1)
    rec_ref[...] = jnp.where(rows == colz, 0.0, r)
    mean_ref[...] = mean
    logvar_ref[...] = logvar


def _blockdiag(att):
    eye = jnp.eye(_HEADS, dtype=att.dtype)
    return (att[:, :, None] * eye[:, None, :]).reshape(_H4, _HEADS)


def kernel(x, adj, W1, att_src1, att_dst1, b1, W2, att_src2, att_dst2, b2,
           fcm_w1, fcm_b1, fcm_w2, fcm_b2, fcl_w1, fcl_b1, fcl_w2, fcl_b2,
           at_w, at_b, dec_w, dec_b):
    eps = jax.random.normal(jax.random.key(42), (_N, _LAT), dtype=jnp.float32)
    adj_recon, mean, logvar = pl.pallas_call(
        _fused_kernel,
        out_shape=(
            jax.ShapeDtypeStruct((_N, _N), jnp.float32),
            jax.ShapeDtypeStruct((_N, _LAT), jnp.float32),
            jax.ShapeDtypeStruct((_N, _LAT), jnp.float32),
        ),
    )(x, adj,
      W1, _blockdiag(att_src1), _blockdiag(att_dst1), b1.reshape(1, _H4),
      W2, _blockdiag(att_src2), _blockdiag(att_dst2), b2.reshape(1, _H4),
      fcm_w1, fcm_b1.reshape(1, _LAT), fcm_w2, fcm_b2.reshape(1, _LAT),
      fcl_w1, fcl_b1.reshape(1, _LAT), fcl_w2, fcl_b2.reshape(1, _LAT),
      at_w, at_b.reshape(1, _HID), eps,
      dec_w[:_HID].reshape(_HID, 1), dec_w[_HID:].reshape(_HID, 1),
      dec_b.reshape(1, 1))
    return adj_recon, mean, logvar


# speedup vs baseline: 5311.7780x; 1.1978x over previous
import jax
import jax.numpy as jnp
from jax.experimental import pallas as pl

_N = 1024
_IN = 256
_HID = 64
_HEADS = 4
_H4 = _HEADS * _HID
_LAT = 64


def _fused_kernel(x_ref, adj_ref,
                  w1_ref, as1_ref, ad1_ref, b1_ref,
                  w2_ref, as2_ref, ad2_ref, b2_ref,
                  fm1_ref, fmb1_ref, fm2_ref, fmb2_ref,
                  fl1_ref, flb1_ref, fl2_ref, flb2_ref,
                  atw_ref, atb_ref, eps_ref, dwa_ref, dwb_ref, db_ref,
                  rec_ref, mean_ref, logvar_ref):
    adjv = adj_ref[...]
    ones_col = jnp.ones((_N, 1), dtype=jnp.float32)

    def gat(inp, w_ref, asw_ref, adw_ref, b_ref):
        h = jnp.dot(inp, w_ref[...], preferred_element_type=jnp.float32)
        asrc = jnp.dot(h, asw_ref[...], preferred_element_type=jnp.float32)
        adstT = jax.lax.dot_general(
            adw_ref[...], h, (((0,), (1,)), ((), ())),
            preferred_element_type=jnp.float32)
        outs = []
        for k in range(_HEADS):
            a = asrc[:, k : k + 1]
            bT = adstT[k : k + 1, :]
            c1 = jnp.max(a)
            c2 = jnp.max(bT)
            half = 0.4 * (c1 + c2)
            ea = jnp.exp(a - c1)
            ebT = jnp.exp(bT - c2)
            ea2 = jnp.exp(0.2 * a - (0.2 * c1 + half))
            eb2T = jnp.exp(0.2 * bT - (0.2 * c2 + half))
            p = adjv * jnp.maximum(ea * ebT, ea2 * eb2T)
            hx = jnp.concatenate(
                [h[:, _HID * k : _HID * (k + 1)], ones_col], axis=1)
            os_ = jax.lax.dot_general(
                p, hx, (((0,), (0,)), ((), ())),
                preferred_element_type=jnp.float32)
            outs.append(os_[:, :_HID] * (1.0 / (os_[:, _HID:] + 1e-16)))
        g = jnp.concatenate(outs, axis=1) + b_ref[...]
        return jnp.maximum(g, 0.0)

    g1 = gat(x_ref[...], w1_ref, as1_ref, ad1_ref, b1_ref)
    g2 = gat(g1, w2_ref, as2_ref, ad2_ref, b2_ref)

    t = jnp.maximum(
        jnp.dot(g2, fm1_ref[...], preferred_element_type=jnp.float32)
        + fmb1_ref[...], 0.0)
    mean = jnp.dot(t, fm2_ref[...], preferred_element_type=jnp.float32) + fmb2_ref[...]
    t2 = jnp.maximum(
        jnp.dot(g2, fl1_ref[...], preferred_element_type=jnp.float32)
        + flb1_ref[...], 0.0)
    logvar = jnp.dot(t2, fl2_ref[...], preferred_element_type=jnp.float32) + flb2_ref[...]
    std = jnp.exp(0.5 * logvar)
    z = mean + eps_ref[...] * std
    zt = jnp.maximum(
        jnp.dot(z, atw_ref[...], preferred_element_type=jnp.float32)
        + atb_ref[...], 0.0)
    u = jnp.dot(zt, dwa_ref[...], preferred_element_type=jnp.float32)
    vT = jax.lax.dot_general(
        dwb_ref[...], zt, (((0,), (1,)), ((), ())),
        preferred_element_type=jnp.float32)
    r = jax.nn.sigmoid(u + vT + db_ref[0, 0])
    rows = jax.lax.broadcasted_iota(jnp.int32, (_N, _N), 0)
    colz = jax.lax.broadcasted_iota(jnp.int32, (_N, _N), 1)
    rec_ref[...] = jnp.where(rows == colz, 0.0, r)
    mean_ref[...] = mean
    logvar_ref[...] = logvar


def _blockdiag(att):
    eye = jnp.eye(_HEADS, dtype=att.dtype)
    return (att[:, :, None] * eye[:, None, :]).reshape(_H4, _HEADS)


def kernel(x, adj, W1, att_src1, att_dst1, b1, W2, att_src2, att_dst2, b2,
           fcm_w1, fcm_b1, fcm_w2, fcm_b2, fcl_w1, fcl_b1, fcl_w2, fcl_b2,
           at_w, at_b, dec_w, dec_b):
    with jax.ensure_compile_time_eval():
        eps = jax.random.normal(jax.random.key(42), (_N, _LAT), dtype=jnp.float32)
    adj_recon, mean, logvar = pl.pallas_call(
        _fused_kernel,
        out_shape=(
            jax.ShapeDtypeStruct((_N, _N), jnp.float32),
            jax.ShapeDtypeStruct((_N, _LAT), jnp.float32),
            jax.ShapeDtypeStruct((_N, _LAT), jnp.float32),
        ),
    )(x, adj,
      W1, _blockdiag(att_src1), _blockdiag(att_dst1), b1.reshape(1, _H4),
      W2, _blockdiag(att_src2), _blockdiag(att_dst2), b2.reshape(1, _H4),
      fcm_w1, fcm_b1.reshape(1, _LAT), fcm_w2, fcm_b2.reshape(1, _LAT),
      fcl_w1, fcl_b1.reshape(1, _LAT), fcl_w2, fcl_b2.reshape(1, _LAT),
      at_w, at_b.reshape(1, _HID), eps,
      dec_w[:_HID].reshape(_HID, 1), dec_w[_HID:].reshape(_HID, 1),
      dec_b.reshape(1, 1))
    return adj_recon, mean, logvar
